# baseline (speedup 1.0000x reference)
import functools

import jax
import jax.numpy as jnp
from jax import lax
from jax.experimental import pallas as pl
from jax.experimental.pallas import tpu as pltpu
from jax.experimental.pallas import tpu_sc as plsc

F = 128
W = 128
NBUF = 2
NT = 16
NC = 2
R = 2000
EPS = 1e-5
F32 = jnp.float32


def _round_up(v, m):
    return (v + m - 1) // m * m


@functools.lru_cache(maxsize=None)
def _make_deg(n_acc, e_pad):
    nwin = e_pad // (NC * NT * W)
    zrows = n_acc // NT
    mesh = plsc.VectorSubcoreMesh(core_axis_name="c", subcore_axis_name="s", num_cores=NC, num_subcores=NT)

    @functools.partial(
        pl.kernel, mesh=mesh,
        out_type=jax.ShapeDtypeStruct((NC * n_acc,), F32),
        scratch_types=[
            pltpu.VMEM((nwin, W), jnp.int32),
            pltpu.VMEM((W,), F32),
            pltpu.VMEM((zrows,), F32),
            pltpu.VMEM_SHARED((n_acc,), F32),
        ],
    )
    def deg_kernel(dst_hbm, out_hbm, dbuf, ones, stage, acc):
        cid = lax.axis_index("c")
        sid = lax.axis_index("s")

        @pl.loop(0, W // 16)
        def _(k):
            ones[pl.ds(k * 16, 16)] = jnp.full((16,), 1.0, F32)

        @pl.loop(0, zrows // 16)
        def _(k):
            stage[pl.ds(k * 16, 16)] = jnp.zeros((16,), F32)

        pltpu.sync_copy(stage, acc.at[pl.ds(sid * zrows, zrows)])
        plsc.subcore_barrier()

        base = (cid * NT + sid) * nwin
        pltpu.sync_copy(dst_hbm.at[pl.ds(base, nwin)], dbuf)

        @pl.loop(0, nwin)
        def _(j):
            pltpu.sync_copy(ones, acc.at[dbuf.at[j]], add=True)

        plsc.subcore_barrier()
        pltpu.sync_copy(acc.at[pl.ds(sid * zrows, zrows)], stage)
        pltpu.sync_copy(stage, out_hbm.at[pl.ds(cid * n_acc + sid * zrows, zrows)])

    return deg_kernel


@functools.lru_cache(maxsize=None)
def _make_agg(n, n_acc, e_pad, nb):
    npc = nb // NC
    nwin = e_pad // (NT * W)
    nseg = 2
    swin = nwin // nseg
    grp = swin // NBUF
    zrows = n_acc // NT
    nzfull = zrows // W
    ztail = zrows % W
    wo_main = (n // (NT * 8)) * 8
    wo_rem = n - NT * wo_main
    wchunk = 48 if wo_main % 48 == 0 else 8
    nwo = wo_main // wchunk
    assert nwin % nseg == 0 and swin % NBUF == 0 and grp >= 1
    assert wo_rem % 8 == 0 and wo_rem <= W and wchunk <= W
    mesh = plsc.VectorSubcoreMesh(core_axis_name="c", subcore_axis_name="s", num_cores=NC, num_subcores=NT)

    @functools.partial(
        pl.kernel, mesh=mesh,
        out_type=jax.ShapeDtypeStruct((nb * n, F), F32),
        scratch_types=(
            [pltpu.VMEM((swin, W), jnp.int32),
             pltpu.VMEM((swin, W), jnp.int32)]
            + [pltpu.VMEM((W, F), F32) for _ in range(NBUF)]
            + [pltpu.VMEM_SHARED((n_acc, F), F32)]
            + [pltpu.SemaphoreType.DMA for _ in range(2 * NBUF)]
        ),
    )
    def agg_kernel(src_hbm, dst_hbm, hp_hbm, out_hbm, sbuf, dbuf, *rest):
        bufs = rest[:NBUF]
        acc = rest[NBUF]
        gsem = rest[NBUF + 1:NBUF + 1 + NBUF]
        ssem = rest[NBUF + 1 + NBUF:]
        cid = lax.axis_index("c")
        sid = lax.axis_index("s")
        bufa = bufs[0]

        def gath(j, b):
            return pltpu.make_async_copy(hp_hbm.at[sbuf.at[j]], bufs[b],
                                         gsem[b])

        def scat(j, b):
            return pltpu.make_async_copy(bufs[b], acc.at[dbuf.at[j]],
                                         ssem[b])

        for q in range(npc):
            p = cid * npc + q
            off = (p * n).astype(jnp.int32)

            @pl.loop(0, W)
            def _(r):
                for k in range(F // 16):
                    bufa[r, pl.ds(k * 16, 16)] = jnp.zeros((16,), F32)

            @pl.loop(0, nzfull)
            def _(z):
                pltpu.sync_copy(bufa, acc.at[pl.ds(sid * zrows + z * W, W)])

            if ztail:
                pltpu.sync_copy(bufa.at[pl.ds(0, ztail)],
                                acc.at[pl.ds(sid * zrows + nzfull * W, ztail)])

            plsc.subcore_barrier()

            for sg in range(nseg):
                wbase = sid * nwin + sg * swin
                pltpu.sync_copy(src_hbm.at[pl.ds(wbase, swin)], sbuf)
                pltpu.sync_copy(dst_hbm.at[pl.ds(wbase, swin)], dbuf)

                @pl.loop(0, swin)
                def _(j):
                    for k in range(W // 16):
                        sl = pl.ds(k * 16, 16)
                        sbuf[j, sl] = sbuf[j, sl] + off

                for b in range(NBUF):
                    gath(b, b).start()

                @pl.loop(0, grp - 1)
                def _(g):
                    j = g * NBUF
                    for b in range(NBUF):
                        gath(j + b, b).wait()
                        scat(j + b, b).start(add=True)
                    for b in range(NBUF):
                        scat(j + b, b).wait()
                        gath(j + NBUF + b, b).start()

                jf = swin - NBUF
                for b in range(NBUF):
                    gath(jf + b, b).wait()
                    scat(jf + b, b).start(add=True)
                for b in range(NBUF):
                    scat(jf + b, b).wait()

            plsc.subcore_barrier()

            @pl.loop(0, nwo)
            def _(t):
                r0 = sid * wo_main + t * wchunk
                pltpu.sync_copy(acc.at[pl.ds(r0, wchunk)],
                                bufa.at[pl.ds(0, wchunk)])
                pltpu.sync_copy(bufa.at[pl.ds(0, wchunk)],
                                out_hbm.at[pl.ds(p * n + r0, wchunk)])

            if wo_rem:
                @pl.when(sid == 0)
                def _():
                    r0 = NT * wo_main
                    pltpu.sync_copy(acc.at[pl.ds(r0, wo_rem)],
                                    bufa.at[pl.ds(0, wo_rem)])
                    pltpu.sync_copy(bufa.at[pl.ds(0, wo_rem)],
                                    out_hbm.at[pl.ds(p * n + r0, wo_rem)])

            plsc.subcore_barrier()

    return agg_kernel


def _tc_mm1(x_ref, w_ref, deg_ref, hp_ref, dinv_ref):
    d = deg_ref[:, 0:1] + deg_ref[:, 1:2] + 1.0
    di = lax.rsqrt(d)
    dinv_ref[...] = di
    hp_ref[...] = di * jnp.dot(x_ref[...], w_ref[...],
                               preferred_element_type=F32)


def _tc_stats(n_rows, rb):
    def body(agg_ref, hp_ref, dinv_ref, b_ref, g_ref, be_ref,
             scale_ref, shift_ref, acc_ref):
        i = pl.program_id(1)
        z = dinv_ref[...] * (agg_ref[...] + hp_ref[...]) + b_ref[0]
        s1 = jnp.sum(z, axis=0, keepdims=True)
        s2 = jnp.sum(z * z, axis=0, keepdims=True)

        @pl.when(i == 0)
        def _():
            acc_ref[0:1] = s1
            acc_ref[1:2] = s2

        @pl.when(i > 0)
        def _():
            acc_ref[0:1] += s1
            acc_ref[1:2] += s2

        @pl.when(i == rb - 1)
        def _():
            mu = acc_ref[0:1] / n_rows
            var = acc_ref[1:2] / n_rows - mu * mu
            sc = g_ref[0] * lax.rsqrt(var + EPS)
            scale_ref[0] = sc
            shift_ref[0] = be_ref[0] - mu * sc

    return body


def _tc_act(agg_ref, hp_ref, dinv_ref, b_ref, scale_ref, shift_ref, out_ref):
    z = dinv_ref[...] * (agg_ref[...] + hp_ref[...]) + b_ref[0]
    out_ref[...] = jnp.maximum(z * scale_ref[0] + shift_ref[0], 0.0)


def _tc_mm(h_ref, w_ref, dinv_ref, hp_ref):
    hp_ref[...] = dinv_ref[...] * jnp.dot(h_ref[...], w_ref[...],
                                          preferred_element_type=F32)


def _tc_final(aggA, aggB, hpA, hpB, dinv_ref, b_ref, out_ref):
    zA = dinv_ref[...] * (aggA[...] + hpA[...]) + b_ref[0:1]
    zB = dinv_ref[...] * (aggB[...] + hpB[...]) + b_ref[1:2]
    z = jnp.concatenate([zA, zB], axis=1)
    m = jnp.max(z, axis=1, keepdims=True)
    lse = m + jnp.log(jnp.sum(jnp.exp(z - m), axis=1, keepdims=True))
    out_ref[...] = z - lse


_ARB = pltpu.CompilerParams(dimension_semantics=("arbitrary", "arbitrary"))


def kernel(x, adj_t, W1, b1, W2, b2, W3, b3, g1, be1, g2, be2):
    n, d_in = x.shape
    d_h = W1.shape[1]
    d_out = W3.shape[1]
    e = adj_t.shape[1]
    nb_h = d_h // F
    nb_o = d_out // F
    rb = n // R

    n_deg = _round_up(n, NT * W)
    n_agg = n + 16
    e_pad = _round_up(e, NT * W * 2 * NBUF)
    pad = e_pad - e

    src = adj_t[0].astype(jnp.int32)
    dst = adj_t[1].astype(jnp.int32)
    fill = jnp.arange(pad, dtype=jnp.int32)
    src_p = jnp.concatenate([src, fill % n]).reshape(e_pad // W, W)
    dst_p = jnp.concatenate([dst, n + fill % 16]).reshape(e_pad // W, W)

    deg2 = _make_deg(n_deg, e_pad)(dst_p)
    degT = jnp.transpose(deg2.reshape(NC, n_deg))[:n]

    b1r = b1.reshape(nb_h, 1, F)
    g1r = g1.reshape(nb_h, 1, F)
    be1r = be1.reshape(nb_h, 1, F)
    b2r = b2.reshape(nb_h, 1, F)
    g2r = g2.reshape(nb_h, 1, F)
    be2r = be2.reshape(nb_h, 1, F)
    b3r = b3.reshape(nb_o, F)

    agg = _make_agg(n, n_agg, e_pad, nb_h)
    agg_o = _make_agg(n, n_agg, e_pad, nb_o) if nb_o != nb_h else agg

    hp1, dinv = pl.pallas_call(
        _tc_mm1,
        grid=(rb, nb_h),
        in_specs=[
            pl.BlockSpec((R, d_in), lambda i, p: (i, 0)),
            pl.BlockSpec((d_in, F), lambda i, p: (0, p)),
            pl.BlockSpec((R, 2), lambda i, p: (i, 0)),
        ],
        out_specs=[
            pl.BlockSpec((R, F), lambda i, p: (p * (n // R) + i, 0)),
            pl.BlockSpec((R, 1), lambda i, p: (i, 0)),
        ],
        out_shape=[
            jax.ShapeDtypeStruct((nb_h * n, F), F32),
            jax.ShapeDtypeStruct((n, 1), F32),
        ],
        compiler_params=_ARB,
    )(x, W1, degT)

    def bn_layer(agg_l, hp_l, b_r, g_r, be_r, w_next, nb_out):
        scale, shift = pl.pallas_call(
            _tc_stats(n, rb),
            grid=(nb_h, rb),
            in_specs=[
                pl.BlockSpec((R, F), lambda p, i: (p * (n // R) + i, 0)),
                pl.BlockSpec((R, F), lambda p, i: (p * (n // R) + i, 0)),
                pl.BlockSpec((R, 1), lambda p, i: (i, 0)),
                pl.BlockSpec((1, 1, F), lambda p, i: (p, 0, 0)),
                pl.BlockSpec((1, 1, F), lambda p, i: (p, 0, 0)),
                pl.BlockSpec((1, 1, F), lambda p, i: (p, 0, 0)),
            ],
            out_specs=[
                pl.BlockSpec((1, 1, F), lambda p, i: (p, 0, 0)),
                pl.BlockSpec((1, 1, F), lambda p, i: (p, 0, 0)),
            ],
            out_shape=[
                jax.ShapeDtypeStruct((nb_h, 1, F), F32),
                jax.ShapeDtypeStruct((nb_h, 1, F), F32),
            ],
            scratch_shapes=[pltpu.VMEM((2, F), F32)],
            compiler_params=_ARB,
        )(agg_l, hp_l, dinv, b_r, g_r, be_r)

        hb = pl.pallas_call(
            _tc_act,
            grid=(nb_h, rb),
            in_specs=[
                pl.BlockSpec((R, F), lambda p, i: (p * (n // R) + i, 0)),
                pl.BlockSpec((R, F), lambda p, i: (p * (n // R) + i, 0)),
                pl.BlockSpec((R, 1), lambda p, i: (i, 0)),
                pl.BlockSpec((1, 1, F), lambda p, i: (p, 0, 0)),
                pl.BlockSpec((1, 1, F), lambda p, i: (p, 0, 0)),
                pl.BlockSpec((1, 1, F), lambda p, i: (p, 0, 0)),
            ],
            out_specs=pl.BlockSpec((R, F), lambda p, i: (i, p)),
            out_shape=jax.ShapeDtypeStruct((n, d_h), F32),
            compiler_params=_ARB,
        )(agg_l, hp_l, dinv, b_r, scale, shift)

        hp_next = pl.pallas_call(
            _tc_mm,
            grid=(rb, nb_out),
            in_specs=[
                pl.BlockSpec((R, d_h), lambda i, p: (i, 0)),
                pl.BlockSpec((d_h, F), lambda i, p: (0, p)),
                pl.BlockSpec((R, 1), lambda i, p: (i, 0)),
            ],
            out_specs=pl.BlockSpec((R, F), lambda i, p: (p * (n // R) + i, 0)),
            out_shape=jax.ShapeDtypeStruct((nb_out * n, F), F32),
            compiler_params=_ARB,
        )(hb, w_next, dinv)
        return hp_next

    agg1 = agg(src_p, dst_p, hp1)
    hp2 = bn_layer(agg1, hp1, b1r, g1r, be1r, W2, nb_h)
    agg2 = agg(src_p, dst_p, hp2)
    hp3 = bn_layer(agg2, hp2, b2r, g2r, be2r, W3, nb_o)
    agg3 = agg_o(src_p, dst_p, hp3)

    out = pl.pallas_call(
        _tc_final,
        grid=(rb,),
        in_specs=[
            pl.BlockSpec((R, F), lambda i: (i, 0)),
            pl.BlockSpec((R, F), lambda i: ((n // R) + i, 0)),
            pl.BlockSpec((R, F), lambda i: (i, 0)),
            pl.BlockSpec((R, F), lambda i: ((n // R) + i, 0)),
            pl.BlockSpec((R, 1), lambda i: (i, 0)),
            pl.BlockSpec((nb_o, F), lambda i: (0, 0)),
        ],
        out_specs=pl.BlockSpec((R, d_out), lambda i: (i, 0)),
        out_shape=jax.ShapeDtypeStruct((n, d_out), F32),
        compiler_params=pltpu.CompilerParams(dimension_semantics=("arbitrary",)),
    )(agg3, agg3, hp3, hp3, dinv, b3r)

    return out


# speedup vs baseline: 1.4222x; 1.4222x over previous
import functools

import jax
import jax.numpy as jnp
from jax import lax
from jax.experimental import pallas as pl
from jax.experimental.pallas import tpu as pltpu
from jax.experimental.pallas import tpu_sc as plsc

F = 128
W = 128
NBUF = 2
NT = 16
NC = 2
R = 2000
EPS = 1e-5
F32 = jnp.float32


def _round_up(v, m):
    return (v + m - 1) // m * m


@functools.lru_cache(maxsize=None)
def _make_deg(n_acc, e_pad):
    nwin = e_pad // (NC * NT * W)
    zrows = n_acc // NT
    mesh = plsc.VectorSubcoreMesh(core_axis_name="c", subcore_axis_name="s", num_cores=NC, num_subcores=NT)

    @functools.partial(
        pl.kernel, mesh=mesh,
        out_type=jax.ShapeDtypeStruct((NC * n_acc,), F32),
        scratch_types=[
            pltpu.VMEM((nwin, W), jnp.int32),
            pltpu.VMEM((W,), F32),
            pltpu.VMEM((zrows,), F32),
            pltpu.VMEM_SHARED((n_acc,), F32),
        ],
    )
    def deg_kernel(dst_hbm, out_hbm, dbuf, ones, stage, acc):
        cid = lax.axis_index("c")
        sid = lax.axis_index("s")

        @pl.loop(0, W // 16)
        def _(k):
            ones[pl.ds(k * 16, 16)] = jnp.full((16,), 1.0, F32)

        @pl.loop(0, zrows // 16)
        def _(k):
            stage[pl.ds(k * 16, 16)] = jnp.zeros((16,), F32)

        pltpu.sync_copy(stage, acc.at[pl.ds(sid * zrows, zrows)])
        plsc.subcore_barrier()

        base = (cid * NT + sid) * nwin
        pltpu.sync_copy(dst_hbm.at[pl.ds(base, nwin)], dbuf)

        @pl.loop(0, nwin)
        def _(j):
            pltpu.sync_copy(ones, acc.at[dbuf.at[j]], add=True)

        plsc.subcore_barrier()
        pltpu.sync_copy(acc.at[pl.ds(sid * zrows, zrows)], stage)
        pltpu.sync_copy(stage, out_hbm.at[pl.ds(cid * n_acc + sid * zrows, zrows)])

    return deg_kernel


@functools.lru_cache(maxsize=None)
def _make_agg(n, n_acc, e_pad, nb):
    npc = nb // NC
    nwin = e_pad // (NT * W)
    nseg = 2
    swin = nwin // nseg
    grp = swin // NBUF
    zrows = n_acc // NT
    nzfull = zrows // W
    ztail = zrows % W
    wo_main = (n // (NT * 8)) * 8
    wo_rem = n - NT * wo_main
    wchunk = 48 if wo_main % 48 == 0 else 8
    nwo = wo_main // wchunk
    assert nwin % nseg == 0 and swin % NBUF == 0 and grp >= 1
    assert wo_rem % 8 == 0 and wo_rem <= W and wchunk <= W
    mesh = plsc.VectorSubcoreMesh(core_axis_name="c", subcore_axis_name="s", num_cores=NC, num_subcores=NT)

    @functools.partial(
        pl.kernel, mesh=mesh,
        out_type=jax.ShapeDtypeStruct((nb * n, F), F32),
        scratch_types=(
            [pltpu.VMEM((swin, W), jnp.int32),
             pltpu.VMEM((swin, W), jnp.int32)]
            + [pltpu.VMEM((W, F), F32) for _ in range(NBUF)]
            + [pltpu.VMEM_SHARED((n_acc, F), F32)]
            + [pltpu.SemaphoreType.DMA for _ in range(2 * NBUF)]
        ),
    )
    def agg_kernel(src_hbm, dst_hbm, hp_hbm, out_hbm, sbuf, dbuf, *rest):
        bufs = rest[:NBUF]
        acc = rest[NBUF]
        gsem = rest[NBUF + 1:NBUF + 1 + NBUF]
        ssem = rest[NBUF + 1 + NBUF:]
        cid = lax.axis_index("c")
        sid = lax.axis_index("s")
        bufa = bufs[0]

        def gath(j, b):
            return pltpu.make_async_copy(hp_hbm.at[sbuf.at[j]], bufs[b],
                                         gsem[b])

        def scat(j, b):
            return pltpu.make_async_copy(bufs[b], acc.at[dbuf.at[j]],
                                         ssem[b])

        for q in range(npc):
            p = cid * npc + q
            off = (p * n).astype(jnp.int32)

            @pl.loop(0, W)
            def _(r):
                for k in range(F // 16):
                    bufa[r, pl.ds(k * 16, 16)] = jnp.zeros((16,), F32)

            @pl.loop(0, nzfull)
            def _(z):
                pltpu.sync_copy(bufa, acc.at[pl.ds(sid * zrows + z * W, W)])

            if ztail:
                pltpu.sync_copy(bufa.at[pl.ds(0, ztail)],
                                acc.at[pl.ds(sid * zrows + nzfull * W, ztail)])

            plsc.subcore_barrier()

            for sg in range(nseg):
                wbase = sid * nwin + sg * swin
                pltpu.sync_copy(src_hbm.at[pl.ds(wbase, swin)], sbuf)
                pltpu.sync_copy(dst_hbm.at[pl.ds(wbase, swin)], dbuf)

                @pl.loop(0, swin)
                def _(j):
                    for k in range(W // 16):
                        sl = pl.ds(k * 16, 16)
                        sbuf[j, sl] = sbuf[j, sl] + off

                gath(0, 0).start()

                @pl.loop(0, grp)
                def _(jj):
                    j0 = jj * 2
                    gath(j0 + 1, 1).start()
                    gath(j0, 0).wait()
                    scat(j0, 0).start(add=True)
                    scat(j0, 0).wait()

                    @pl.when(jj < grp - 1)
                    def _():
                        gath(j0 + 2, 0).start()

                    gath(j0 + 1, 1).wait()
                    scat(j0 + 1, 1).start(add=True)
                    scat(j0 + 1, 1).wait()

            plsc.subcore_barrier()

            @pl.loop(0, nwo)
            def _(t):
                r0 = sid * wo_main + t * wchunk
                pltpu.sync_copy(acc.at[pl.ds(r0, wchunk)],
                                bufa.at[pl.ds(0, wchunk)])
                pltpu.sync_copy(bufa.at[pl.ds(0, wchunk)],
                                out_hbm.at[pl.ds(p * n + r0, wchunk)])

            if wo_rem:
                @pl.when(sid == 0)
                def _():
                    r0 = NT * wo_main
                    pltpu.sync_copy(acc.at[pl.ds(r0, wo_rem)],
                                    bufa.at[pl.ds(0, wo_rem)])
                    pltpu.sync_copy(bufa.at[pl.ds(0, wo_rem)],
                                    out_hbm.at[pl.ds(p * n + r0, wo_rem)])

            plsc.subcore_barrier()

    return agg_kernel


def _mk_mm1(nb):
    def body(x_ref, w_ref, deg_ref, hp_ref, dinv_ref):
        d = deg_ref[:, 0:1] + deg_ref[:, 1:2] + 1.0
        di = lax.rsqrt(d)
        dinv_ref[...] = di
        h = jnp.dot(x_ref[...], w_ref[...], preferred_element_type=F32)
        for p in range(nb):
            hp_ref[p] = di * h[:, p * F:(p + 1) * F]
    return body


def _mk_mid(nb, nb_out, rb, n_rows):
    def body(agg_ref, hp_ref, dinv_ref, b_ref, g_ref, be_ref, w_ref,
             hpn_ref, st_ref):
        ph = pl.program_id(0)
        i = pl.program_id(1)
        di = dinv_ref[...]

        def zblk(p):
            return di * (agg_ref[p] + hp_ref[p]) + b_ref[p]

        @pl.when(ph == 0)
        def _():
            z = jnp.concatenate([zblk(p) for p in range(nb)], axis=1)
            s1 = jnp.sum(z, axis=0, keepdims=True)
            s2 = jnp.sum(z * z, axis=0, keepdims=True)

            @pl.when(i == 0)
            def _():
                st_ref[0:1] = s1
                st_ref[1:2] = s2

            @pl.when(i > 0)
            def _():
                st_ref[0:1] += s1
                st_ref[1:2] += s2

            @pl.when(i == rb - 1)
            def _():
                mu = st_ref[0:1] / n_rows
                var = st_ref[1:2] / n_rows - mu * mu
                g = jnp.concatenate([g_ref[p] for p in range(nb)], axis=1)
                be = jnp.concatenate([be_ref[p] for p in range(nb)], axis=1)
                sc = g * lax.rsqrt(var + EPS)
                st_ref[2:3] = sc
                st_ref[3:4] = be - mu * sc

        @pl.when(ph == 1)
        def _():
            z = jnp.concatenate([zblk(p) for p in range(nb)], axis=1)
            a = jnp.maximum(z * st_ref[2:3] + st_ref[3:4], 0.0)
            h = jnp.dot(a, w_ref[...], preferred_element_type=F32)
            for p in range(nb_out):
                hpn_ref[p] = di * h[:, p * F:(p + 1) * F]
    return body


def _mk_final(nb):
    def body(agg_ref, hp_ref, dinv_ref, b_ref, out_ref):
        di = dinv_ref[...]
        z = jnp.concatenate(
            [di * (agg_ref[p] + hp_ref[p]) + b_ref[p] for p in range(nb)],
            axis=1)
        m = jnp.max(z, axis=1, keepdims=True)
        lse = m + jnp.log(jnp.sum(jnp.exp(z - m), axis=1, keepdims=True))
        out_ref[...] = z - lse
    return body


_ARB1 = pltpu.CompilerParams(dimension_semantics=("arbitrary",))
_ARB2 = pltpu.CompilerParams(dimension_semantics=("arbitrary", "arbitrary"))


def kernel(x, adj_t, W1, b1, W2, b2, W3, b3, g1, be1, g2, be2):
    n, d_in = x.shape
    d_h = W1.shape[1]
    d_out = W3.shape[1]
    e = adj_t.shape[1]
    nb_h = d_h // F
    nb_o = d_out // F
    rb = n // R

    n_deg = _round_up(n, NT * W)
    n_agg = n + 16
    e_pad = _round_up(e, NT * W * 2 * NBUF)
    pad = e_pad - e

    src = adj_t[0].astype(jnp.int32)
    dst = adj_t[1].astype(jnp.int32)
    fill = jnp.arange(pad, dtype=jnp.int32)
    src_p = jnp.concatenate([src, fill % n]).reshape(e_pad // W, W)
    dst_p = jnp.concatenate([dst, n + fill % 16]).reshape(e_pad // W, W)

    deg2 = _make_deg(n_deg, e_pad)(dst_p)
    degT = jnp.transpose(deg2.reshape(NC, n_deg))[:n]

    b1r = b1.reshape(nb_h, 1, F)
    g1r = g1.reshape(nb_h, 1, F)
    be1r = be1.reshape(nb_h, 1, F)
    b2r = b2.reshape(nb_h, 1, F)
    g2r = g2.reshape(nb_h, 1, F)
    be2r = be2.reshape(nb_h, 1, F)
    b3r = b3.reshape(nb_o, 1, F)

    agg = _make_agg(n, n_agg, e_pad, nb_h)
    agg_o = _make_agg(n, n_agg, e_pad, nb_o) if nb_o != nb_h else agg

    blk = lambda nb: pl.BlockSpec((nb, R, F), lambda *g: (0, g[-1], 0))
    vec = pl.BlockSpec((R, 1), lambda *g: (g[-1], 0))
    par = lambda nb: pl.BlockSpec((nb, 1, F), lambda *g: (0, 0, 0))

    hp1, dinv = pl.pallas_call(
        _mk_mm1(nb_h),
        grid=(rb,),
        in_specs=[
            pl.BlockSpec((R, d_in), lambda i: (i, 0)),
            pl.BlockSpec((d_in, d_h), lambda i: (0, 0)),
            pl.BlockSpec((R, 2), lambda i: (i, 0)),
        ],
        out_specs=[blk(nb_h), vec],
        out_shape=[
            jax.ShapeDtypeStruct((nb_h, n, F), F32),
            jax.ShapeDtypeStruct((n, 1), F32),
        ],
        compiler_params=_ARB1,
    )(x, W1, degT)

    def mid_layer(agg_l, hp_l, b_r, g_r, be_r, w_next, nb_out):
        return pl.pallas_call(
            _mk_mid(nb_h, nb_out, rb, n),
            grid=(2, rb),
            in_specs=[
                blk(nb_h), blk(nb_h), vec,
                par(nb_h), par(nb_h), par(nb_h),
                pl.BlockSpec((d_h, F * nb_out), lambda ph, i: (0, 0)),
            ],
            out_specs=pl.BlockSpec((nb_out, R, F),
                                   lambda ph, i: (0, i * ph, 0)),
            out_shape=jax.ShapeDtypeStruct((nb_out, n, F), F32),
            scratch_shapes=[pltpu.VMEM((4, d_h), F32)],
            compiler_params=_ARB2,
        )(agg_l, hp_l, dinv, b_r, g_r, be_r, w_next)

    def to2d(a):
        return a.reshape(a.shape[0] * a.shape[1], F)

    def to3d(a, nb):
        return a.reshape(nb, n, F)

    agg1 = to3d(agg(src_p, dst_p, to2d(hp1)), nb_h)
    hp2 = mid_layer(agg1, hp1, b1r, g1r, be1r, W2, nb_h)
    agg2 = to3d(agg(src_p, dst_p, to2d(hp2)), nb_h)
    hp3 = mid_layer(agg2, hp2, b2r, g2r, be2r, W3, nb_o)
    agg3 = to3d(agg_o(src_p, dst_p, to2d(hp3)), nb_o)

    out = pl.pallas_call(
        _mk_final(nb_o),
        grid=(rb,),
        in_specs=[blk(nb_o), blk(nb_o), vec, par(nb_o)],
        out_specs=pl.BlockSpec((R, d_out), lambda i: (i, 0)),
        out_shape=jax.ShapeDtypeStruct((n, d_out), F32),
        compiler_params=_ARB1,
    )(agg3, hp3, dinv, b3r)

    return out


# speedup vs baseline: 1.4646x; 1.0298x over previous
import functools

import jax
import jax.numpy as jnp
from jax import lax
from jax.experimental import pallas as pl
from jax.experimental.pallas import tpu as pltpu
from jax.experimental.pallas import tpu_sc as plsc

F = 128
W = 128
NBUF = 2
NT = 16
NC = 2
R = 2000
EPS = 1e-5
F32 = jnp.float32


def _round_up(v, m):
    return (v + m - 1) // m * m


@functools.lru_cache(maxsize=None)
def _make_deg(n_acc, e_pad):
    nwin = e_pad // (NC * NT * W)
    zrows = n_acc // NT
    mesh = plsc.VectorSubcoreMesh(core_axis_name="c", subcore_axis_name="s", num_cores=NC, num_subcores=NT)

    @functools.partial(
        pl.kernel, mesh=mesh,
        out_type=jax.ShapeDtypeStruct((NC * n_acc,), F32),
        scratch_types=[
            pltpu.VMEM((nwin, W), jnp.int32),
            pltpu.VMEM((W,), F32),
            pltpu.VMEM((zrows,), F32),
            pltpu.VMEM_SHARED((n_acc,), F32),
        ],
    )
    def deg_kernel(dst_hbm, out_hbm, dbuf, ones, stage, acc):
        cid = lax.axis_index("c")
        sid = lax.axis_index("s")

        @pl.loop(0, W // 16)
        def _(k):
            ones[pl.ds(k * 16, 16)] = jnp.full((16,), 1.0, F32)

        @pl.loop(0, zrows // 16)
        def _(k):
            stage[pl.ds(k * 16, 16)] = jnp.zeros((16,), F32)

        pltpu.sync_copy(stage, acc.at[pl.ds(sid * zrows, zrows)])
        plsc.subcore_barrier()

        base = (cid * NT + sid) * nwin
        pltpu.sync_copy(dst_hbm.at[pl.ds(base, nwin)], dbuf)

        @pl.loop(0, nwin)
        def _(j):
            pltpu.sync_copy(ones, acc.at[dbuf.at[j]], add=True)

        plsc.subcore_barrier()
        pltpu.sync_copy(acc.at[pl.ds(sid * zrows, zrows)], stage)
        pltpu.sync_copy(stage, out_hbm.at[pl.ds(cid * n_acc + sid * zrows, zrows)])

    return deg_kernel


@functools.lru_cache(maxsize=None)
def _make_agg(n, n_acc, e_pad, nb):
    npc = nb // NC
    nwin = e_pad // (NT * W)
    nseg = 2
    swin = nwin // nseg
    grp = swin // NBUF
    zrows = n_acc // NT
    nzfull = zrows // W
    ztail = zrows % W
    wo_main = (n // (NT * 8)) * 8
    wo_rem = n - NT * wo_main
    wchunk = 48 if wo_main % 48 == 0 else 8
    nwo = wo_main // wchunk
    assert nwin % nseg == 0 and swin % NBUF == 0 and grp >= 1
    assert wo_rem % 8 == 0 and wo_rem <= W and wchunk <= W
    mesh = plsc.VectorSubcoreMesh(core_axis_name="c", subcore_axis_name="s", num_cores=NC, num_subcores=NT)

    @functools.partial(
        pl.kernel, mesh=mesh,
        out_type=jax.ShapeDtypeStruct((nb * n, F), F32),
        scratch_types=(
            [pltpu.VMEM((swin, W), jnp.int32),
             pltpu.VMEM((swin, W), jnp.int32)]
            + [pltpu.VMEM((W, F), F32) for _ in range(NBUF)]
            + [pltpu.VMEM_SHARED((n_acc, F), F32)]
            + [pltpu.SemaphoreType.DMA for _ in range(2 * NBUF)]
        ),
    )
    def agg_kernel(src_hbm, dst_hbm, hp_hbm, out_hbm, sbuf, dbuf, *rest):
        bufs = rest[:NBUF]
        acc = rest[NBUF]
        gsem = rest[NBUF + 1:NBUF + 1 + NBUF]
        ssem = rest[NBUF + 1 + NBUF:]
        cid = lax.axis_index("c")
        sid = lax.axis_index("s")
        bufa = bufs[0]

        def gath(j, b):
            return pltpu.make_async_copy(hp_hbm.at[sbuf.at[j]], bufs[b],
                                         gsem[b])

        def scat(j, b):
            return pltpu.make_async_copy(bufs[b], acc.at[dbuf.at[j]],
                                         ssem[b])

        nwrows = e_pad // W
        for q in range(npc):
            p = cid * npc + q

            @pl.loop(0, W)
            def _(r):
                for k in range(F // 16):
                    bufa[r, pl.ds(k * 16, 16)] = jnp.zeros((16,), F32)

            @pl.loop(0, nzfull)
            def _(z):
                pltpu.sync_copy(bufa, acc.at[pl.ds(sid * zrows + z * W, W)])

            if ztail:
                pltpu.sync_copy(bufa.at[pl.ds(0, ztail)],
                                acc.at[pl.ds(sid * zrows + nzfull * W, ztail)])

            plsc.subcore_barrier()

            for sg in range(nseg):
                wbase = sid * nwin + sg * swin
                pltpu.sync_copy(src_hbm.at[pl.ds(p * nwrows + wbase, swin)],
                                sbuf)
                pltpu.sync_copy(dst_hbm.at[pl.ds(wbase, swin)], dbuf)

                gath(0, 0).start()

                @pl.loop(0, grp)
                def _(jj):
                    j0 = jj * 2
                    gath(j0 + 1, 1).start()
                    gath(j0, 0).wait()
                    scat(j0, 0).start(add=True)
                    scat(j0, 0).wait()

                    @pl.when(jj < grp - 1)
                    def _():
                        gath(j0 + 2, 0).start()

                    gath(j0 + 1, 1).wait()
                    scat(j0 + 1, 1).start(add=True)
                    scat(j0 + 1, 1).wait()

            plsc.subcore_barrier()

            @pl.loop(0, nwo)
            def _(t):
                r0 = sid * wo_main + t * wchunk
                pltpu.sync_copy(acc.at[pl.ds(r0, wchunk)],
                                bufa.at[pl.ds(0, wchunk)])
                pltpu.sync_copy(bufa.at[pl.ds(0, wchunk)],
                                out_hbm.at[pl.ds(p * n + r0, wchunk)])

            if wo_rem:
                @pl.when(sid == 0)
                def _():
                    r0 = NT * wo_main
                    pltpu.sync_copy(acc.at[pl.ds(r0, wo_rem)],
                                    bufa.at[pl.ds(0, wo_rem)])
                    pltpu.sync_copy(bufa.at[pl.ds(0, wo_rem)],
                                    out_hbm.at[pl.ds(p * n + r0, wo_rem)])

            plsc.subcore_barrier()

    return agg_kernel


def _mk_mm1(nb):
    def body(x_ref, w_ref, deg_ref, hp_ref, dinv_ref):
        d = deg_ref[:, 0:1] + deg_ref[:, 1:2] + 1.0
        di = lax.rsqrt(d)
        dinv_ref[...] = di
        h = jnp.dot(x_ref[...], w_ref[...], preferred_element_type=F32)
        for p in range(nb):
            hp_ref[p] = di * h[:, p * F:(p + 1) * F]
    return body


def _mk_mid(nb, nb_out, rb, n_rows):
    def body(agg_ref, hp_ref, dinv_ref, b_ref, g_ref, be_ref, w_ref,
             hpn_ref, st_ref, z_ref):
        ph = pl.program_id(0)
        i = pl.program_id(1)
        di = dinv_ref[...]

        @pl.when(ph == 0)
        def _():
            z = jnp.concatenate(
                [di * (agg_ref[p] + hp_ref[p]) + b_ref[p] for p in range(nb)],
                axis=1)
            z_ref[i] = z
            s1 = jnp.sum(z, axis=0, keepdims=True)
            s2 = jnp.sum(z * z, axis=0, keepdims=True)

            @pl.when(i == 0)
            def _():
                st_ref[0:1] = s1
                st_ref[1:2] = s2

            @pl.when(i > 0)
            def _():
                st_ref[0:1] += s1
                st_ref[1:2] += s2

            @pl.when(i == rb - 1)
            def _():
                mu = st_ref[0:1] / n_rows
                var = st_ref[1:2] / n_rows - mu * mu
                g = jnp.concatenate([g_ref[p] for p in range(nb)], axis=1)
                be = jnp.concatenate([be_ref[p] for p in range(nb)], axis=1)
                sc = g * lax.rsqrt(var + EPS)
                st_ref[2:3] = sc
                st_ref[3:4] = be - mu * sc

        @pl.when(ph == 1)
        def _():
            a = jnp.maximum(z_ref[i] * st_ref[2:3] + st_ref[3:4], 0.0)
            h = jnp.dot(a, w_ref[...], preferred_element_type=F32)
            for p in range(nb_out):
                hpn_ref[p] = di * h[:, p * F:(p + 1) * F]
    return body


def _mk_final(nb):
    def body(agg_ref, hp_ref, dinv_ref, b_ref, out_ref):
        di = dinv_ref[...]
        z = jnp.concatenate(
            [di * (agg_ref[p] + hp_ref[p]) + b_ref[p] for p in range(nb)],
            axis=1)
        m = jnp.max(z, axis=1, keepdims=True)
        lse = m + jnp.log(jnp.sum(jnp.exp(z - m), axis=1, keepdims=True))
        out_ref[...] = z - lse
    return body


_ARB1 = pltpu.CompilerParams(dimension_semantics=("arbitrary",))
_ARB2 = pltpu.CompilerParams(dimension_semantics=("arbitrary", "arbitrary"))


def kernel(x, adj_t, W1, b1, W2, b2, W3, b3, g1, be1, g2, be2):
    n, d_in = x.shape
    d_h = W1.shape[1]
    d_out = W3.shape[1]
    e = adj_t.shape[1]
    nb_h = d_h // F
    nb_o = d_out // F
    rb = n // R

    n_deg = _round_up(n, NT * W)
    n_agg = n + 16
    e_pad = _round_up(e, NT * W * 2 * NBUF)
    pad = e_pad - e

    src = adj_t[0].astype(jnp.int32)
    dst = adj_t[1].astype(jnp.int32)
    fill = jnp.arange(pad, dtype=jnp.int32)
    src_e = jnp.concatenate([src, fill % n])
    src_p = (src_e[None, :] +
             (jnp.arange(nb_h, dtype=jnp.int32) * n)[:, None]
             ).reshape(nb_h * (e_pad // W), W)
    dst_p = jnp.concatenate([dst, n + fill % 16]).reshape(e_pad // W, W)

    deg2 = _make_deg(n_deg, e_pad)(dst_p)
    degT = jnp.transpose(deg2.reshape(NC, n_deg))[:n]

    b1r = b1.reshape(nb_h, 1, F)
    g1r = g1.reshape(nb_h, 1, F)
    be1r = be1.reshape(nb_h, 1, F)
    b2r = b2.reshape(nb_h, 1, F)
    g2r = g2.reshape(nb_h, 1, F)
    be2r = be2.reshape(nb_h, 1, F)
    b3r = b3.reshape(nb_o, 1, F)

    agg = _make_agg(n, n_agg, e_pad, nb_h)
    agg_o = _make_agg(n, n_agg, e_pad, nb_o) if nb_o != nb_h else agg

    blk = lambda nb: pl.BlockSpec((nb, R, F), lambda *g: (0, g[-1], 0))
    blk0 = lambda nb: pl.BlockSpec((nb, R, F),
                                   lambda ph, i: (0, i * (1 - ph), 0))
    vec = pl.BlockSpec((R, 1), lambda *g: (g[-1], 0))
    par = lambda nb: pl.BlockSpec((nb, 1, F), lambda *g: (0, 0, 0))

    hp1, dinv = pl.pallas_call(
        _mk_mm1(nb_h),
        grid=(rb,),
        in_specs=[
            pl.BlockSpec((R, d_in), lambda i: (i, 0)),
            pl.BlockSpec((d_in, d_h), lambda i: (0, 0)),
            pl.BlockSpec((R, 2), lambda i: (i, 0)),
        ],
        out_specs=[blk(nb_h), vec],
        out_shape=[
            jax.ShapeDtypeStruct((nb_h, n, F), F32),
            jax.ShapeDtypeStruct((n, 1), F32),
        ],
        compiler_params=_ARB1,
    )(x, W1, degT)

    def mid_layer(agg_l, hp_l, b_r, g_r, be_r, w_next, nb_out):
        return pl.pallas_call(
            _mk_mid(nb_h, nb_out, rb, n),
            grid=(2, rb),
            in_specs=[
                blk0(nb_h), blk0(nb_h), vec,
                par(nb_h), par(nb_h), par(nb_h),
                pl.BlockSpec((d_h, F * nb_out), lambda ph, i: (0, 0)),
            ],
            out_specs=pl.BlockSpec((nb_out, R, F),
                                   lambda ph, i: (0, i * ph, 0)),
            out_shape=jax.ShapeDtypeStruct((nb_out, n, F), F32),
            scratch_shapes=[pltpu.VMEM((4, d_h), F32),
                            pltpu.VMEM((rb, R, d_h), F32)],
            compiler_params=_ARB2,
        )(agg_l, hp_l, dinv, b_r, g_r, be_r, w_next)

    def to2d(a):
        return a.reshape(a.shape[0] * a.shape[1], F)

    def to3d(a, nb):
        return a.reshape(nb, n, F)

    agg1 = to3d(agg(src_p, dst_p, to2d(hp1)), nb_h)
    hp2 = mid_layer(agg1, hp1, b1r, g1r, be1r, W2, nb_h)
    agg2 = to3d(agg(src_p, dst_p, to2d(hp2)), nb_h)
    hp3 = mid_layer(agg2, hp2, b2r, g2r, be2r, W3, nb_o)
    agg3 = to3d(agg_o(src_p, dst_p, to2d(hp3)), nb_o)

    out = pl.pallas_call(
        _mk_final(nb_o),
        grid=(rb,),
        in_specs=[blk(nb_o), blk(nb_o), vec, par(nb_o)],
        out_specs=pl.BlockSpec((R, d_out), lambda i: (i, 0)),
        out_shape=jax.ShapeDtypeStruct((n, d_out), F32),
        compiler_params=_ARB1,
    )(agg3, hp3, dinv, b3r)

    return out


# speedup vs baseline: 1.4832x; 1.0127x over previous
import functools

import jax
import jax.numpy as jnp
from jax import lax
from jax.experimental import pallas as pl
from jax.experimental.pallas import tpu as pltpu
from jax.experimental.pallas import tpu_sc as plsc

F = 128
W = 128
NBUF = 2
NT = 16
NC = 2
R = 2000
EPS = 1e-5
F32 = jnp.float32


def _round_up(v, m):
    return (v + m - 1) // m * m


@functools.lru_cache(maxsize=None)
def _make_deg(n_acc, e_pad):
    nwin = e_pad // (NC * NT * W)
    zrows = n_acc // NT
    mesh = plsc.VectorSubcoreMesh(core_axis_name="c", subcore_axis_name="s", num_cores=NC, num_subcores=NT)

    @functools.partial(
        pl.kernel, mesh=mesh,
        out_type=jax.ShapeDtypeStruct((NC * n_acc,), F32),
        scratch_types=[
            pltpu.VMEM((nwin, W), jnp.int32),
            pltpu.VMEM((W,), F32),
            pltpu.VMEM((zrows,), F32),
            pltpu.VMEM_SHARED((n_acc,), F32),
        ],
    )
    def deg_kernel(dst_hbm, out_hbm, dbuf, ones, stage, acc):
        cid = lax.axis_index("c")
        sid = lax.axis_index("s")

        @pl.loop(0, W // 16)
        def _(k):
            ones[pl.ds(k * 16, 16)] = jnp.full((16,), 1.0, F32)

        @pl.loop(0, zrows // 16)
        def _(k):
            stage[pl.ds(k * 16, 16)] = jnp.zeros((16,), F32)

        pltpu.sync_copy(stage, acc.at[pl.ds(sid * zrows, zrows)])
        plsc.subcore_barrier()

        base = (cid * NT + sid) * nwin
        pltpu.sync_copy(dst_hbm.at[pl.ds(base, nwin)], dbuf)

        @pl.loop(0, nwin)
        def _(j):
            pltpu.sync_copy(ones, acc.at[dbuf.at[j]], add=True)

        plsc.subcore_barrier()
        pltpu.sync_copy(acc.at[pl.ds(sid * zrows, zrows)], stage)
        pltpu.sync_copy(stage, out_hbm.at[pl.ds(cid * n_acc + sid * zrows, zrows)])

    return deg_kernel


@functools.lru_cache(maxsize=None)
def _make_agg(n, n_acc, e_pad, nb):
    npc = nb // NC
    nwin = e_pad // (NT * W)
    nseg = 2
    swin = nwin // nseg
    grp = swin // NBUF
    zrows = n_acc // NT
    nzfull = zrows // W
    ztail = zrows % W
    wo_main = (n // (NT * 8)) * 8
    wo_rem = n - NT * wo_main
    wchunk = 48 if wo_main % 48 == 0 else 8
    nwo = wo_main // wchunk
    assert nwin % nseg == 0 and swin % NBUF == 0 and grp >= 1
    assert wo_rem % 8 == 0 and wo_rem <= W and wchunk <= W
    mesh = plsc.VectorSubcoreMesh(core_axis_name="c", subcore_axis_name="s", num_cores=NC, num_subcores=NT)

    @functools.partial(
        pl.kernel, mesh=mesh,
        out_type=jax.ShapeDtypeStruct((nb * n, F), F32),
        scratch_types=(
            [pltpu.VMEM((swin, W), jnp.int32),
             pltpu.VMEM((swin, W), jnp.int32)]
            + [pltpu.VMEM((W, F), F32) for _ in range(NBUF)]
            + [pltpu.VMEM_SHARED((n_acc, F), F32)]
            + [pltpu.SemaphoreType.DMA for _ in range(2 * NBUF)]
        ),
    )
    def agg_kernel(src_hbm, dst_hbm, hp_hbm, out_hbm, sbuf, dbuf, *rest):
        bufs = rest[:NBUF]
        acc = rest[NBUF]
        gsem = rest[NBUF + 1:NBUF + 1 + NBUF]
        ssem = rest[NBUF + 1 + NBUF:]
        cid = lax.axis_index("c")
        sid = lax.axis_index("s")
        bufa = bufs[0]

        def gath(j, b):
            return pltpu.make_async_copy(hp_hbm.at[sbuf.at[j]], bufs[b],
                                         gsem[b])

        def scat(j, b):
            return pltpu.make_async_copy(bufs[b], acc.at[dbuf.at[j]],
                                         ssem[b])

        nwrows = e_pad // W

        def load_idx(p, sg):
            wbase = sid * nwin + sg * swin
            pltpu.sync_copy(src_hbm.at[pl.ds(p * nwrows + wbase, swin)],
                            sbuf)
            pltpu.sync_copy(dst_hbm.at[pl.ds(wbase, swin)], dbuf)

        bufb = bufs[1]
        for q in range(npc):
            p = cid * npc + q

            load_idx(p, 0)
            gath(0, 0).start()

            @pl.loop(0, W)
            def _(r):
                for k in range(F // 16):
                    bufb[r, pl.ds(k * 16, 16)] = jnp.zeros((16,), F32)

            @pl.loop(0, nzfull)
            def _(z):
                pltpu.sync_copy(bufb, acc.at[pl.ds(sid * zrows + z * W, W)])

            if ztail:
                pltpu.sync_copy(bufb.at[pl.ds(0, ztail)],
                                acc.at[pl.ds(sid * zrows + nzfull * W, ztail)])

            plsc.subcore_barrier()

            for sg in range(nseg):
                if sg:
                    load_idx(p, sg)
                    gath(0, 0).start()

                @pl.loop(0, grp)
                def _(jj):
                    j0 = jj * 2
                    gath(j0 + 1, 1).start()
                    gath(j0, 0).wait()
                    scat(j0, 0).start(add=True)
                    scat(j0, 0).wait()

                    @pl.when(jj < grp - 1)
                    def _():
                        gath(j0 + 2, 0).start()

                    gath(j0 + 1, 1).wait()
                    scat(j0 + 1, 1).start(add=True)
                    scat(j0 + 1, 1).wait()

            plsc.subcore_barrier()

            @pl.loop(0, nwo)
            def _(t):
                r0 = sid * wo_main + t * wchunk
                pltpu.sync_copy(acc.at[pl.ds(r0, wchunk)],
                                bufa.at[pl.ds(0, wchunk)])
                pltpu.sync_copy(bufa.at[pl.ds(0, wchunk)],
                                out_hbm.at[pl.ds(p * n + r0, wchunk)])

            if wo_rem:
                @pl.when(sid == 0)
                def _():
                    r0 = NT * wo_main
                    pltpu.sync_copy(acc.at[pl.ds(r0, wo_rem)],
                                    bufa.at[pl.ds(0, wo_rem)])
                    pltpu.sync_copy(bufa.at[pl.ds(0, wo_rem)],
                                    out_hbm.at[pl.ds(p * n + r0, wo_rem)])

            plsc.subcore_barrier()

    return agg_kernel


def _mk_mm1(nb):
    def body(x_ref, w_ref, deg_ref, hp_ref, dinv_ref):
        d = deg_ref[:, 0:1] + deg_ref[:, 1:2] + 1.0
        di = lax.rsqrt(d)
        dinv_ref[...] = di
        h = jnp.dot(x_ref[...], w_ref[...], preferred_element_type=F32)
        for p in range(nb):
            hp_ref[p] = di * h[:, p * F:(p + 1) * F]
    return body


def _mk_mid(nb, nb_out, rb, n_rows):
    def body(agg_ref, hp_ref, dinv_ref, b_ref, g_ref, be_ref, w_ref,
             hpn_ref, st_ref, z_ref):
        ph = pl.program_id(0)
        i = pl.program_id(1)
        di = dinv_ref[...]

        @pl.when(ph == 0)
        def _():
            z = jnp.concatenate(
                [di * (agg_ref[p] + hp_ref[p]) + b_ref[p] for p in range(nb)],
                axis=1)
            z_ref[i] = z
            s1 = jnp.sum(z, axis=0, keepdims=True)
            s2 = jnp.sum(z * z, axis=0, keepdims=True)

            @pl.when(i == 0)
            def _():
                st_ref[0:1] = s1
                st_ref[1:2] = s2

            @pl.when(i > 0)
            def _():
                st_ref[0:1] += s1
                st_ref[1:2] += s2

            @pl.when(i == rb - 1)
            def _():
                mu = st_ref[0:1] / n_rows
                var = st_ref[1:2] / n_rows - mu * mu
                g = jnp.concatenate([g_ref[p] for p in range(nb)], axis=1)
                be = jnp.concatenate([be_ref[p] for p in range(nb)], axis=1)
                sc = g * lax.rsqrt(var + EPS)
                st_ref[2:3] = sc
                st_ref[3:4] = be - mu * sc

        @pl.when(ph == 1)
        def _():
            a = jnp.maximum(z_ref[i] * st_ref[2:3] + st_ref[3:4], 0.0)
            h = jnp.dot(a, w_ref[...], preferred_element_type=F32)
            for p in range(nb_out):
                hpn_ref[p] = di * h[:, p * F:(p + 1) * F]
    return body


def _mk_final(nb):
    def body(agg_ref, hp_ref, dinv_ref, b_ref, out_ref):
        di = dinv_ref[...]
        z = jnp.concatenate(
            [di * (agg_ref[p] + hp_ref[p]) + b_ref[p] for p in range(nb)],
            axis=1)
        m = jnp.max(z, axis=1, keepdims=True)
        lse = m + jnp.log(jnp.sum(jnp.exp(z - m), axis=1, keepdims=True))
        out_ref[...] = z - lse
    return body


_ARB1 = pltpu.CompilerParams(dimension_semantics=("arbitrary",))
_ARB2 = pltpu.CompilerParams(dimension_semantics=("arbitrary", "arbitrary"))


def kernel(x, adj_t, W1, b1, W2, b2, W3, b3, g1, be1, g2, be2):
    n, d_in = x.shape
    d_h = W1.shape[1]
    d_out = W3.shape[1]
    e = adj_t.shape[1]
    nb_h = d_h // F
    nb_o = d_out // F
    rb = n // R

    n_deg = _round_up(n, NT * W)
    n_agg = n + 16
    e_pad = _round_up(e, NT * W * 2 * NBUF)
    pad = e_pad - e

    src = adj_t[0].astype(jnp.int32)
    dst = adj_t[1].astype(jnp.int32)
    fill = jnp.arange(pad, dtype=jnp.int32)
    src_e = jnp.concatenate([src, fill % n])
    src_p = (src_e[None, :] +
             (jnp.arange(nb_h, dtype=jnp.int32) * n)[:, None]
             ).reshape(nb_h * (e_pad // W), W)
    dst_p = jnp.concatenate([dst, n + fill % 16]).reshape(e_pad // W, W)

    deg2 = _make_deg(n_deg, e_pad)(dst_p)
    degT = jnp.transpose(deg2.reshape(NC, n_deg))[:n]

    b1r = b1.reshape(nb_h, 1, F)
    g1r = g1.reshape(nb_h, 1, F)
    be1r = be1.reshape(nb_h, 1, F)
    b2r = b2.reshape(nb_h, 1, F)
    g2r = g2.reshape(nb_h, 1, F)
    be2r = be2.reshape(nb_h, 1, F)
    b3r = b3.reshape(nb_o, 1, F)

    agg = _make_agg(n, n_agg, e_pad, nb_h)
    agg_o = _make_agg(n, n_agg, e_pad, nb_o) if nb_o != nb_h else agg

    blk = lambda nb: pl.BlockSpec((nb, R, F), lambda *g: (0, g[-1], 0))
    blk0 = lambda nb: pl.BlockSpec((nb, R, F),
                                   lambda ph, i: (0, i * (1 - ph), 0))
    vec = pl.BlockSpec((R, 1), lambda *g: (g[-1], 0))
    par = lambda nb: pl.BlockSpec((nb, 1, F), lambda *g: (0, 0, 0))

    hp1, dinv = pl.pallas_call(
        _mk_mm1(nb_h),
        grid=(rb,),
        in_specs=[
            pl.BlockSpec((R, d_in), lambda i: (i, 0)),
            pl.BlockSpec((d_in, d_h), lambda i: (0, 0)),
            pl.BlockSpec((R, 2), lambda i: (i, 0)),
        ],
        out_specs=[blk(nb_h), vec],
        out_shape=[
            jax.ShapeDtypeStruct((nb_h, n, F), F32),
            jax.ShapeDtypeStruct((n, 1), F32),
        ],
        compiler_params=_ARB1,
    )(x, W1, degT)

    def mid_layer(agg_l, hp_l, b_r, g_r, be_r, w_next, nb_out):
        return pl.pallas_call(
            _mk_mid(nb_h, nb_out, rb, n),
            grid=(2, rb),
            in_specs=[
                blk0(nb_h), blk0(nb_h), vec,
                par(nb_h), par(nb_h), par(nb_h),
                pl.BlockSpec((d_h, F * nb_out), lambda ph, i: (0, 0)),
            ],
            out_specs=pl.BlockSpec((nb_out, R, F),
                                   lambda ph, i: (0, i * ph, 0)),
            out_shape=jax.ShapeDtypeStruct((nb_out, n, F), F32),
            scratch_shapes=[pltpu.VMEM((4, d_h), F32),
                            pltpu.VMEM((rb, R, d_h), F32)],
            compiler_params=_ARB2,
        )(agg_l, hp_l, dinv, b_r, g_r, be_r, w_next)

    def to2d(a):
        return a.reshape(a.shape[0] * a.shape[1], F)

    def to3d(a, nb):
        return a.reshape(nb, n, F)

    agg1 = to3d(agg(src_p, dst_p, to2d(hp1)), nb_h)
    hp2 = mid_layer(agg1, hp1, b1r, g1r, be1r, W2, nb_h)
    agg2 = to3d(agg(src_p, dst_p, to2d(hp2)), nb_h)
    hp3 = mid_layer(agg2, hp2, b2r, g2r, be2r, W3, nb_o)
    agg3 = to3d(agg_o(src_p, dst_p, to2d(hp3)), nb_o)

    out = pl.pallas_call(
        _mk_final(nb_o),
        grid=(rb,),
        in_specs=[blk(nb_o), blk(nb_o), vec, par(nb_o)],
        out_specs=pl.BlockSpec((R, d_out), lambda i: (i, 0)),
        out_shape=jax.ShapeDtypeStruct((n, d_out), F32),
        compiler_params=_ARB1,
    )(agg3, hp3, dinv, b3r)

    return out
